# Initial kernel scaffold; baseline (speedup 1.0000x reference)
import jax
import jax.numpy as jnp
from jax.experimental import pallas as pl


def kernel(x, edge_index, edge_feats, edge_types, W_iou_w, W_iou_b, U_iou_w, U_iou_b, W_f_w, W_f_b, U_f_w, U_f_b):
    raise NotImplementedError("write your pallas kernel here")


# speedup vs baseline: 5.0459x; 5.0459x over previous
import jax
import jax.numpy as jnp
from jax import lax
from jax.experimental import pallas as pl
from jax.experimental.pallas import tpu as pltpu
from jax.experimental.pallas import tpu_sc as plsc

N = 10000
E = 160000
F = 128
F3 = 384
NP = 10240
TRASH = N
LCAP = 8192
PE = E + 16 * LCAP
TILE = 256
NC = 2
NS = 16
NW = NC * NS
CH = 16


def _sigmoid(v):
    return 1.0 / (1.0 + jnp.exp(-v))


def _a1_body(mes, mee, schpos, sppos, csidx, xwf, ufb,
             c_st, huf_st, hui_st, hsu_st, fcb_st, cnew_st,
             vmeta, idxC, idxP, idxS,
             bufC, bufCP, bufHUf, bufXWf, bufHUi, bufhsU, buf_fc, buf_cn,
             bufb, sem0, sem1, sem2, sem3, sem4, sem5):
    wid = lax.axis_index("s") * NC + lax.axis_index("c")
    pltpu.sync_copy(mes, vmeta)
    es = vmeta[...][0]
    pltpu.sync_copy(mee, vmeta)
    ee = vmeta[...][0]
    pltpu.sync_copy(ufb, bufb)
    nr = (ee - es + (CH * NW - 1)) // (CH * NW)

    def round_body(r, carry):
        start = pl.multiple_of(es + (r * NW + wid) * CH, CH)

        @pl.when(start < ee)
        def _():
            pltpu.sync_copy(schpos.at[pl.ds(start, CH)], idxC)
            pltpu.sync_copy(sppos.at[pl.ds(start, CH)], idxP)
            pltpu.sync_copy(csidx.at[pl.ds(start, CH)], idxS)
            g0 = pltpu.make_async_copy(c_st.at[idxC], bufC, sem0)
            g1 = pltpu.make_async_copy(huf_st.at[idxC], bufHUf, sem1)
            g2 = pltpu.make_async_copy(hui_st.at[idxC], bufHUi, sem2)
            g3 = pltpu.make_async_copy(xwf.at[idxP], bufXWf, sem3)
            g4 = pltpu.make_async_copy(hsu_st.at[idxP], bufhsU, sem4)
            g5 = pltpu.make_async_copy(c_st.at[idxP], bufCP, sem5)
            g0.start(); g1.start(); g2.start(); g3.start(); g4.start(); g5.start()
            g0.wait(); g1.wait(); g2.wait(); g3.wait(); g4.wait(); g5.wait()

            def row_body(i, c2):
                for j in range(F // 16):
                    sl = pl.ds(j * 16, 16)
                    s = bufXWf[i, sl] + bufHUf[i, sl] + bufb[sl]
                    f = 1.0 / (1.0 + jnp.exp(-s))
                    fc = f * bufC[i, sl]
                    buf_fc[i, sl] = fc
                    buf_cn[i, sl] = bufCP[i, sl] + fc
                for j in range(F3 // 16):
                    sl = pl.ds(j * 16, 16)
                    bufhsU[i, sl] = bufhsU[i, sl] + bufHUi[i, sl]
                return c2

            lax.fori_loop(0, CH, row_body, 0)
            s0 = pltpu.make_async_copy(bufhsU, hsu_st.at[idxP], sem0)
            s1 = pltpu.make_async_copy(buf_fc, fcb_st.at[idxP], sem1)
            s2 = pltpu.make_async_copy(buf_cn, cnew_st.at[idxS], sem2)
            s0.start(); s1.start(); s2.start()
            s0.wait(); s1.wait(); s2.wait()
        return carry

    lax.fori_loop(0, nr, round_body, 0)


def _a2_body(mes, mee, csidx, cnew_st, c_st,
             vmeta, idxS, buf, sem0):
    wid = lax.axis_index("s") * NC + lax.axis_index("c")
    pltpu.sync_copy(mes, vmeta)
    es = vmeta[...][0]
    pltpu.sync_copy(mee, vmeta)
    ee = vmeta[...][0]
    nr = (ee - es + (CH * NW - 1)) // (CH * NW)

    def round_body(r, carry):
        start = pl.multiple_of(es + (r * NW + wid) * CH, CH)

        @pl.when(start < ee)
        def _():
            pltpu.sync_copy(csidx.at[pl.ds(start, CH)], idxS)
            g = pltpu.make_async_copy(cnew_st.at[idxS], buf, sem0)
            g.start(); g.wait()
            s = pltpu.make_async_copy(buf, c_st.at[idxS], sem0)
            s.start(); s.wait()
        return carry

    lax.fori_loop(0, nr, round_body, 0)


def _b_body(meta, uiouw, ufw, biou, xw,
            hsu_st, fcb_st, c_st, h_st, hui_st, huf_st,
            meta_s, uiouw_v, ufw_v, biou_v,
            xw_v, hsu_v, fc_v, c_v, h_v, hui_v, huf_v,
            s0, s1, s2, s3, s4, s5, s6):
    cp = pltpu.make_async_copy(meta, meta_s, s0)
    cp.start(); cp.wait()
    cw0 = pltpu.make_async_copy(uiouw, uiouw_v, s0)
    cw1 = pltpu.make_async_copy(ufw, ufw_v, s1)
    cw2 = pltpu.make_async_copy(biou, biou_v, s2)
    cw0.start(); cw1.start(); cw2.start()
    cw0.wait(); cw1.wait(); cw2.wait()
    ns = meta_s[0]
    ne = meta_s[1]
    lvl = meta_s[2]
    ts = ns // TILE
    te = (ne + TILE - 1) // TILE

    def tile_body(t, carry):
        r0 = t * TILE
        sl = pl.ds(r0, TILE)
        g0 = pltpu.make_async_copy(xw.at[sl], xw_v, s0)
        g1 = pltpu.make_async_copy(hsu_st.at[sl], hsu_v, s1)
        g2 = pltpu.make_async_copy(fcb_st.at[sl], fc_v, s2)
        g3 = pltpu.make_async_copy(c_st.at[sl], c_v, s3)
        g4 = pltpu.make_async_copy(h_st.at[sl], h_v, s4)
        g5 = pltpu.make_async_copy(hui_st.at[sl], hui_v, s5)
        g6 = pltpu.make_async_copy(huf_st.at[sl], huf_v, s6)
        g0.start(); g1.start(); g2.start(); g3.start(); g4.start(); g5.start(); g6.start()
        g0.wait(); g1.wait(); g2.wait(); g3.wait(); g4.wait(); g5.wait(); g6.wait()

        rowid = r0 + lax.broadcasted_iota(jnp.int32, (TILE, 1), 0)
        active = (rowid >= ns) & (rowid < ne)
        use_u = (lvl > 0).astype(jnp.float32)
        iou = xw_v[...] + use_u * (hsu_v[...] + biou_v[...])
        i_g = _sigmoid(iou[:, :F])
        o_g = _sigmoid(iou[:, F:2 * F])
        u_g = jnp.tanh(iou[:, 2 * F:])
        cn = i_g * u_g + use_u * fc_v[...]
        hn = o_g * jnp.tanh(cn)
        hui_n = lax.dot_general(hn, uiouw_v[...], (((1,), (1,)), ((), ())),
                                preferred_element_type=jnp.float32)
        huf_n = lax.dot_general(hn, ufw_v[...], (((1,), (1,)), ((), ())),
                                preferred_element_type=jnp.float32)
        c_v[...] = jnp.where(active, cn, c_v[...])
        h_v[...] = jnp.where(active, hn, h_v[...])
        hui_v[...] = jnp.where(active, hui_n, hui_v[...])
        huf_v[...] = jnp.where(active, huf_n, huf_v[...])
        w0 = pltpu.make_async_copy(c_v, c_st.at[sl], s3)
        w1 = pltpu.make_async_copy(h_v, h_st.at[sl], s4)
        w2 = pltpu.make_async_copy(hui_v, hui_st.at[sl], s5)
        w3 = pltpu.make_async_copy(huf_v, huf_st.at[sl], s6)
        w0.start(); w1.start(); w2.start(); w3.start()
        w0.wait(); w1.wait(); w2.wait(); w3.wait()
        return carry

    lax.fori_loop(ts, te, tile_body, 0)


def _pre_body(x_ref, wiou_ref, wf_ref, biou_ref, bf_ref, xwiou_ref, xwf_ref):
    xv = x_ref[...]
    xwiou_ref[...] = lax.dot_general(
        xv, wiou_ref[...], (((1,), (1,)), ((), ())),
        preferred_element_type=jnp.float32) + biou_ref[...]
    xwf_ref[...] = lax.dot_general(
        xv, wf_ref[...], (((1,), (1,)), ((), ())),
        preferred_element_type=jnp.float32) + bf_ref[...]


def kernel(x, edge_index, edge_feats, edge_types, W_iou_w, W_iou_b,
           U_iou_w, U_iou_b, W_f_w, W_f_b, U_f_w, U_f_b):
    del edge_feats, edge_types
    i32 = jnp.int32
    f32 = jnp.float32
    p = edge_index[0]
    ch = edge_index[1]

    deg = jnp.zeros((N,), i32).at[p].add(1)
    order = jnp.argsort(p, stable=True)
    sp0 = p[order]
    eidx = jnp.arange(E, dtype=i32)
    is_start = jnp.concatenate(
        [jnp.zeros((1,), bool), sp0[1:] != sp0[:-1]])
    gs = lax.cummax(jnp.where(is_start, eidx, 0), axis=0)
    edge_order = jnp.zeros((E,), i32).at[order].set(eidx - gs + 1)

    sp = p
    so = edge_order
    ecnt = jnp.zeros((E + 2,), i32).at[edge_order].add(1)
    eoff = jnp.concatenate([jnp.zeros((1,), i32), jnp.cumsum(ecnt)])
    pc = ((ecnt + 15) // 16) * 16
    pc = jnp.where(ecnt > 0, pc, 0)
    peoff = jnp.concatenate([jnp.zeros((1,), i32), jnp.cumsum(pc)])
    nperm = jnp.argsort(deg, stable=True)
    ncnt = jnp.zeros((E + 2,), i32).at[deg].add(1)
    noff = jnp.concatenate([jnp.zeros((1,), i32), jnp.cumsum(ncnt)])
    spos = jnp.zeros((N,), i32).at[nperm].set(jnp.arange(N, dtype=i32))

    eperm = jnp.argsort(edge_order, stable=True)
    so_sorted = edge_order[eperm]
    rank_sorted = jnp.arange(E, dtype=i32) - eoff[so_sorted]
    pidx_sorted = peoff[so_sorted] + rank_sorted
    pidx = jnp.zeros((E,), i32).at[eperm].set(pidx_sorted)

    fin = deg[sp] == so
    schpos = jnp.full((PE,), TRASH, i32).at[pidx].set(spos[ch])
    sppos = jnp.full((PE,), TRASH, i32).at[pidx].set(spos[sp])
    csidx = jnp.full((PE,), TRASH, i32).at[pidx].set(
        jnp.where(fin, TRASH, spos[sp]))
    n_iter = jnp.max(deg) + 1

    x_s = jnp.zeros((NP, F), f32).at[:N].set(x[nperm])
    biou2 = W_iou_b.reshape(1, F3)
    bf2 = W_f_b.reshape(1, F)
    xw_iou_s, xw_f_s = pl.pallas_call(
        _pre_body,
        grid=(NP // TILE,),
        in_specs=[
            pl.BlockSpec((TILE, F), lambda t: (t, 0)),
            pl.BlockSpec((F3, F), lambda t: (0, 0)),
            pl.BlockSpec((F, F), lambda t: (0, 0)),
            pl.BlockSpec((1, F3), lambda t: (0, 0)),
            pl.BlockSpec((1, F), lambda t: (0, 0)),
        ],
        out_specs=[
            pl.BlockSpec((TILE, F3), lambda t: (t, 0)),
            pl.BlockSpec((TILE, F), lambda t: (t, 0)),
        ],
        out_shape=[
            jax.ShapeDtypeStruct((NP, F3), f32),
            jax.ShapeDtypeStruct((NP, F), f32),
        ],
    )(x_s, W_iou_w, W_f_w, biou2, bf2)

    c_ref = jax.new_ref(jnp.zeros((NP, F), f32))
    hsu_ref = jax.new_ref(jnp.zeros((NP, F3), f32))
    hui_ref = jax.new_ref(jnp.zeros((NP, F3), f32))
    huf_ref = jax.new_ref(jnp.zeros((NP, F), f32))
    h_ref = jax.new_ref(jnp.zeros((NP, F), f32))
    fcb_ref = jax.new_ref(jnp.zeros((NP, F), f32))
    cnew_ref = jax.new_ref(jnp.zeros((NP, F), f32))

    sc_mesh = plsc.VectorSubcoreMesh(core_axis_name="c", subcore_axis_name="s")
    tc_mesh = pltpu.create_tensorcore_mesh("x")

    a1_call = pl.kernel(
        _a1_body,
        mesh=sc_mesh,
        out_type=[],
        scratch_types=[
            pltpu.VMEM((16,), i32),
            pltpu.VMEM((CH,), i32),
            pltpu.VMEM((CH,), i32),
            pltpu.VMEM((CH,), i32),
            pltpu.VMEM((CH, F), f32),
            pltpu.VMEM((CH, F), f32),
            pltpu.VMEM((CH, F), f32),
            pltpu.VMEM((CH, F), f32),
            pltpu.VMEM((CH, F3), f32),
            pltpu.VMEM((CH, F3), f32),
            pltpu.VMEM((CH, F), f32),
            pltpu.VMEM((CH, F), f32),
            pltpu.VMEM((F,), f32),
            pltpu.SemaphoreType.DMA,
            pltpu.SemaphoreType.DMA,
            pltpu.SemaphoreType.DMA,
            pltpu.SemaphoreType.DMA,
            pltpu.SemaphoreType.DMA,
            pltpu.SemaphoreType.DMA,
        ],
    )

    a2_call = pl.kernel(
        _a2_body,
        mesh=sc_mesh,
        out_type=[],
        scratch_types=[
            pltpu.VMEM((16,), i32),
            pltpu.VMEM((CH,), i32),
            pltpu.VMEM((CH, F), f32),
            pltpu.SemaphoreType.DMA,
        ],
    )

    b_call = pl.kernel(
        _b_body,
        mesh=tc_mesh,
        out_type=[],
        scratch_types=[
            pltpu.SMEM((3,), i32),
            pltpu.VMEM((F3, F), f32),
            pltpu.VMEM((F, F), f32),
            pltpu.VMEM((1, F3), f32),
            pltpu.VMEM((TILE, F3), f32),
            pltpu.VMEM((TILE, F3), f32),
            pltpu.VMEM((TILE, F), f32),
            pltpu.VMEM((TILE, F), f32),
            pltpu.VMEM((TILE, F), f32),
            pltpu.VMEM((TILE, F3), f32),
            pltpu.VMEM((TILE, F), f32),
            pltpu.SemaphoreType.DMA,
            pltpu.SemaphoreType.DMA,
            pltpu.SemaphoreType.DMA,
            pltpu.SemaphoreType.DMA,
            pltpu.SemaphoreType.DMA,
            pltpu.SemaphoreType.DMA,
            pltpu.SemaphoreType.DMA,
        ],
    )

    ufb1 = U_f_b.reshape(F)
    biou1 = U_iou_b.reshape(1, F3)

    def body(n, carry):
        es = peoff[n]
        ee = peoff[n + 1]
        ns = noff[n]
        ne = noff[n + 1]
        mes = jnp.full((16,), es, i32)
        mee = jnp.full((16,), ee, i32)
        a1_call(mes, mee, schpos, sppos, csidx, xw_f_s, ufb1,
                c_ref, huf_ref, hui_ref, hsu_ref, fcb_ref, cnew_ref)
        a2_call(mes, mee, csidx, cnew_ref, c_ref)
        meta = jnp.stack([ns, ne, n]).astype(i32)
        b_call(meta, U_iou_w, U_f_w, biou1, xw_iou_s,
               hsu_ref, fcb_ref, c_ref, h_ref, hui_ref, huf_ref)
        return carry

    lax.fori_loop(0, n_iter, body, 0)
    return h_ref[...][spos]


# speedup vs baseline: 5.4581x; 1.0817x over previous
import jax
import jax.numpy as jnp
from jax import lax
from jax.experimental import pallas as pl
from jax.experimental.pallas import tpu as pltpu
from jax.experimental.pallas import tpu_sc as plsc

N = 10000
E = 160000
F = 128
F3 = 384
NP = 10240
TRASH = N
LCAP = 8192
CH = 64
PE = E + CH * LCAP
TILE = 256
NC = 2
NS = 16
NW = NC * NS


def _sigmoid(v):
    return 1.0 / (1.0 + jnp.exp(-v))


def _a1_body(mes, mee, schpos, sppos, csidx, xwf, ufb,
             c_st, huf_st, hui_st, hsu_st, fcb_st, cnew_st,
             vmeta, idxC, idxP, idxS,
             bufC, bufCP, bufHUf, bufXWf, bufHUi, bufhsU, buf_fc, buf_cn,
             bufb, sem0, sem1, sem2, sem3, sem4, sem5):
    wid = lax.axis_index("s") * NC + lax.axis_index("c")
    pltpu.sync_copy(mes, vmeta)
    es = vmeta[...][0]
    pltpu.sync_copy(mee, vmeta)
    ee = vmeta[...][0]
    pltpu.sync_copy(ufb, bufb)
    nr = (ee - es + (CH * NW - 1)) // (CH * NW)

    def round_body(r, carry):
        start = pl.multiple_of(es + (r * NW + wid) * CH, CH)

        @pl.when(start < ee)
        def _():
            pltpu.sync_copy(schpos.at[pl.ds(start, CH)], idxC)
            pltpu.sync_copy(sppos.at[pl.ds(start, CH)], idxP)
            pltpu.sync_copy(csidx.at[pl.ds(start, CH)], idxS)
            g0 = pltpu.make_async_copy(c_st.at[idxC], bufC, sem0)
            g1 = pltpu.make_async_copy(huf_st.at[idxC], bufHUf, sem1)
            g2 = pltpu.make_async_copy(hui_st.at[idxC], bufHUi, sem2)
            g3 = pltpu.make_async_copy(xwf.at[idxP], bufXWf, sem3)
            g4 = pltpu.make_async_copy(hsu_st.at[idxP], bufhsU, sem4)
            g5 = pltpu.make_async_copy(c_st.at[idxP], bufCP, sem5)
            g0.start(); g1.start(); g2.start(); g3.start(); g4.start(); g5.start()
            g0.wait(); g1.wait(); g2.wait(); g3.wait(); g4.wait(); g5.wait()

            def row_body(i, c2):
                for j in range(F // 16):
                    sl = pl.ds(j * 16, 16)
                    s = bufXWf[i, sl] + bufHUf[i, sl] + bufb[sl]
                    f = 1.0 / (1.0 + jnp.exp(-s))
                    fc = f * bufC[i, sl]
                    buf_fc[i, sl] = fc
                    buf_cn[i, sl] = bufCP[i, sl] + fc
                for j in range(F3 // 16):
                    sl = pl.ds(j * 16, 16)
                    bufhsU[i, sl] = bufhsU[i, sl] + bufHUi[i, sl]
                return c2

            lax.fori_loop(0, CH, row_body, 0)
            s0 = pltpu.make_async_copy(bufhsU, hsu_st.at[idxP], sem0)
            s1 = pltpu.make_async_copy(buf_fc, fcb_st.at[idxP], sem1)
            s2 = pltpu.make_async_copy(buf_cn, cnew_st.at[idxS], sem2)
            s0.start(); s1.start(); s2.start()
            s0.wait(); s1.wait(); s2.wait()
        return carry

    lax.fori_loop(0, nr, round_body, 0)


def _a2_body(mes, mee, csidx, cnew_st, c_st,
             vmeta, idxS, buf, sem0):
    wid = lax.axis_index("s") * NC + lax.axis_index("c")
    pltpu.sync_copy(mes, vmeta)
    es = vmeta[...][0]
    pltpu.sync_copy(mee, vmeta)
    ee = vmeta[...][0]
    nr = (ee - es + (CH * NW - 1)) // (CH * NW)

    def round_body(r, carry):
        start = pl.multiple_of(es + (r * NW + wid) * CH, CH)

        @pl.when(start < ee)
        def _():
            pltpu.sync_copy(csidx.at[pl.ds(start, CH)], idxS)
            g = pltpu.make_async_copy(cnew_st.at[idxS], buf, sem0)
            g.start(); g.wait()
            s = pltpu.make_async_copy(buf, c_st.at[idxS], sem0)
            s.start(); s.wait()
        return carry

    lax.fori_loop(0, nr, round_body, 0)


def _b_body(meta, uiouw, ufw, biou, xw,
            hsu_st, fcb_st, c_st, h_st, hui_st, huf_st,
            meta_s, uiouw_v, ufw_v, biou_v,
            xw_v, hsu_v, fc_v, c_v, h_v, hui_v, huf_v,
            s0, s1, s2, s3, s4, s5, s6):
    cp = pltpu.make_async_copy(meta, meta_s, s0)
    cp.start(); cp.wait()
    cw0 = pltpu.make_async_copy(uiouw, uiouw_v, s0)
    cw1 = pltpu.make_async_copy(ufw, ufw_v, s1)
    cw2 = pltpu.make_async_copy(biou, biou_v, s2)
    cw0.start(); cw1.start(); cw2.start()
    cw0.wait(); cw1.wait(); cw2.wait()
    ns = meta_s[0]
    ne = meta_s[1]
    lvl = meta_s[2]
    ts = ns // TILE
    te = (ne + TILE - 1) // TILE

    def tile_body(t, carry):
        r0 = t * TILE
        sl = pl.ds(r0, TILE)
        g0 = pltpu.make_async_copy(xw.at[sl], xw_v, s0)
        g1 = pltpu.make_async_copy(hsu_st.at[sl], hsu_v, s1)
        g2 = pltpu.make_async_copy(fcb_st.at[sl], fc_v, s2)
        g3 = pltpu.make_async_copy(c_st.at[sl], c_v, s3)
        g4 = pltpu.make_async_copy(h_st.at[sl], h_v, s4)
        g5 = pltpu.make_async_copy(hui_st.at[sl], hui_v, s5)
        g6 = pltpu.make_async_copy(huf_st.at[sl], huf_v, s6)
        g0.start(); g1.start(); g2.start(); g3.start(); g4.start(); g5.start(); g6.start()
        g0.wait(); g1.wait(); g2.wait(); g3.wait(); g4.wait(); g5.wait(); g6.wait()

        rowid = r0 + lax.broadcasted_iota(jnp.int32, (TILE, 1), 0)
        active = (rowid >= ns) & (rowid < ne)
        use_u = (lvl > 0).astype(jnp.float32)
        iou = xw_v[...] + use_u * (hsu_v[...] + biou_v[...])
        i_g = _sigmoid(iou[:, :F])
        o_g = _sigmoid(iou[:, F:2 * F])
        u_g = jnp.tanh(iou[:, 2 * F:])
        cn = i_g * u_g + use_u * fc_v[...]
        hn = o_g * jnp.tanh(cn)
        hui_n = lax.dot_general(hn, uiouw_v[...], (((1,), (1,)), ((), ())),
                                preferred_element_type=jnp.float32)
        huf_n = lax.dot_general(hn, ufw_v[...], (((1,), (1,)), ((), ())),
                                preferred_element_type=jnp.float32)
        c_v[...] = jnp.where(active, cn, c_v[...])
        h_v[...] = jnp.where(active, hn, h_v[...])
        hui_v[...] = jnp.where(active, hui_n, hui_v[...])
        huf_v[...] = jnp.where(active, huf_n, huf_v[...])
        w0 = pltpu.make_async_copy(c_v, c_st.at[sl], s3)
        w1 = pltpu.make_async_copy(h_v, h_st.at[sl], s4)
        w2 = pltpu.make_async_copy(hui_v, hui_st.at[sl], s5)
        w3 = pltpu.make_async_copy(huf_v, huf_st.at[sl], s6)
        w0.start(); w1.start(); w2.start(); w3.start()
        w0.wait(); w1.wait(); w2.wait(); w3.wait()
        return carry

    lax.fori_loop(ts, te, tile_body, 0)


def _pre_body(x_ref, wiou_ref, wf_ref, biou_ref, bf_ref, xwiou_ref, xwf_ref):
    xv = x_ref[...]
    xwiou_ref[...] = lax.dot_general(
        xv, wiou_ref[...], (((1,), (1,)), ((), ())),
        preferred_element_type=jnp.float32) + biou_ref[...]
    xwf_ref[...] = lax.dot_general(
        xv, wf_ref[...], (((1,), (1,)), ((), ())),
        preferred_element_type=jnp.float32) + bf_ref[...]


def kernel(x, edge_index, edge_feats, edge_types, W_iou_w, W_iou_b,
           U_iou_w, U_iou_b, W_f_w, W_f_b, U_f_w, U_f_b):
    del edge_feats, edge_types
    i32 = jnp.int32
    f32 = jnp.float32
    p = edge_index[0]
    ch = edge_index[1]

    deg = jnp.zeros((N,), i32).at[p].add(1)
    order = jnp.argsort(p, stable=True)
    sp0 = p[order]
    eidx = jnp.arange(E, dtype=i32)
    is_start = jnp.concatenate(
        [jnp.zeros((1,), bool), sp0[1:] != sp0[:-1]])
    gs = lax.cummax(jnp.where(is_start, eidx, 0), axis=0)
    edge_order = jnp.zeros((E,), i32).at[order].set(eidx - gs + 1)

    sp = p
    so = edge_order
    ecnt = jnp.zeros((E + 2,), i32).at[edge_order].add(1)
    pc = ((ecnt + CH - 1) // CH) * CH
    pc = jnp.where(ecnt > 0, pc, 0)
    peoff = jnp.concatenate([jnp.zeros((1,), i32), jnp.cumsum(pc)])
    nperm = jnp.argsort(deg, stable=True)
    ncnt = jnp.zeros((E + 2,), i32).at[deg].add(1)
    noff = jnp.concatenate([jnp.zeros((1,), i32), jnp.cumsum(ncnt)])
    spos = jnp.zeros((N,), i32).at[nperm].set(jnp.arange(N, dtype=i32))

    pidx = peoff[so] + (spos[sp] - noff[so])

    fin = deg[sp] == so
    schpos = jnp.full((PE,), TRASH, i32).at[pidx].set(spos[ch])
    sppos = jnp.full((PE,), TRASH, i32).at[pidx].set(spos[sp])
    csidx = jnp.full((PE,), TRASH, i32).at[pidx].set(
        jnp.where(fin, TRASH, spos[sp]))
    n_iter = jnp.max(deg) + 1

    x_s = jnp.zeros((NP, F), f32).at[:N].set(x[nperm])
    biou2 = W_iou_b.reshape(1, F3)
    bf2 = W_f_b.reshape(1, F)
    xw_iou_s, xw_f_s = pl.pallas_call(
        _pre_body,
        grid=(NP // TILE,),
        in_specs=[
            pl.BlockSpec((TILE, F), lambda t: (t, 0)),
            pl.BlockSpec((F3, F), lambda t: (0, 0)),
            pl.BlockSpec((F, F), lambda t: (0, 0)),
            pl.BlockSpec((1, F3), lambda t: (0, 0)),
            pl.BlockSpec((1, F), lambda t: (0, 0)),
        ],
        out_specs=[
            pl.BlockSpec((TILE, F3), lambda t: (t, 0)),
            pl.BlockSpec((TILE, F), lambda t: (t, 0)),
        ],
        out_shape=[
            jax.ShapeDtypeStruct((NP, F3), f32),
            jax.ShapeDtypeStruct((NP, F), f32),
        ],
    )(x_s, W_iou_w, W_f_w, biou2, bf2)

    c_ref = jax.new_ref(jnp.zeros((NP, F), f32))
    hsu_ref = jax.new_ref(jnp.zeros((NP, F3), f32))
    hui_ref = jax.new_ref(jnp.zeros((NP, F3), f32))
    huf_ref = jax.new_ref(jnp.zeros((NP, F), f32))
    h_ref = jax.new_ref(jnp.zeros((NP, F), f32))
    fcb_ref = jax.new_ref(jnp.zeros((NP, F), f32))
    cnew_ref = jax.new_ref(jnp.zeros((NP, F), f32))

    sc_mesh = plsc.VectorSubcoreMesh(core_axis_name="c", subcore_axis_name="s")
    tc_mesh = pltpu.create_tensorcore_mesh("x")

    a1_call = pl.kernel(
        _a1_body,
        mesh=sc_mesh,
        out_type=[],
        scratch_types=[
            pltpu.VMEM((16,), i32),
            pltpu.VMEM((CH,), i32),
            pltpu.VMEM((CH,), i32),
            pltpu.VMEM((CH,), i32),
            pltpu.VMEM((CH, F), f32),
            pltpu.VMEM((CH, F), f32),
            pltpu.VMEM((CH, F), f32),
            pltpu.VMEM((CH, F), f32),
            pltpu.VMEM((CH, F3), f32),
            pltpu.VMEM((CH, F3), f32),
            pltpu.VMEM((CH, F), f32),
            pltpu.VMEM((CH, F), f32),
            pltpu.VMEM((F,), f32),
            pltpu.SemaphoreType.DMA,
            pltpu.SemaphoreType.DMA,
            pltpu.SemaphoreType.DMA,
            pltpu.SemaphoreType.DMA,
            pltpu.SemaphoreType.DMA,
            pltpu.SemaphoreType.DMA,
        ],
    )

    a2_call = pl.kernel(
        _a2_body,
        mesh=sc_mesh,
        out_type=[],
        scratch_types=[
            pltpu.VMEM((16,), i32),
            pltpu.VMEM((CH,), i32),
            pltpu.VMEM((CH, F), f32),
            pltpu.SemaphoreType.DMA,
        ],
    )

    b_call = pl.kernel(
        _b_body,
        mesh=tc_mesh,
        out_type=[],
        scratch_types=[
            pltpu.SMEM((3,), i32),
            pltpu.VMEM((F3, F), f32),
            pltpu.VMEM((F, F), f32),
            pltpu.VMEM((1, F3), f32),
            pltpu.VMEM((TILE, F3), f32),
            pltpu.VMEM((TILE, F3), f32),
            pltpu.VMEM((TILE, F), f32),
            pltpu.VMEM((TILE, F), f32),
            pltpu.VMEM((TILE, F), f32),
            pltpu.VMEM((TILE, F3), f32),
            pltpu.VMEM((TILE, F), f32),
            pltpu.SemaphoreType.DMA,
            pltpu.SemaphoreType.DMA,
            pltpu.SemaphoreType.DMA,
            pltpu.SemaphoreType.DMA,
            pltpu.SemaphoreType.DMA,
            pltpu.SemaphoreType.DMA,
            pltpu.SemaphoreType.DMA,
        ],
    )

    ufb1 = U_f_b.reshape(F)
    biou1 = U_iou_b.reshape(1, F3)

    def body(n, carry):
        es = peoff[n]
        ee = peoff[n + 1]
        ns = noff[n]
        ne = noff[n + 1]
        mes = jnp.full((16,), es, i32)
        mee = jnp.full((16,), ee, i32)
        a1_call(mes, mee, schpos, sppos, csidx, xw_f_s, ufb1,
                c_ref, huf_ref, hui_ref, hsu_ref, fcb_ref, cnew_ref)
        a2_call(mes, mee, csidx, cnew_ref, c_ref)
        meta = jnp.stack([ns, ne, n]).astype(i32)
        b_call(meta, U_iou_w, U_f_w, biou1, xw_iou_s,
               hsu_ref, fcb_ref, c_ref, h_ref, hui_ref, huf_ref)
        return carry

    lax.fori_loop(0, n_iter, body, 0)
    return h_ref[...][spos]


# speedup vs baseline: 6.0779x; 1.1136x over previous
import jax
import jax.numpy as jnp
from jax import lax
from jax.experimental import pallas as pl
from jax.experimental.pallas import tpu as pltpu
from jax.experimental.pallas import tpu_sc as plsc

N = 10000
E = 160000
F = 128
F3 = 384
NP = 10240
TRASH = N
LCAP = 8192
CH = 64
PE = E + CH * LCAP
TILE = 256
NC = 2
NS = 16
NW = NC * NS


def _sigmoid(v):
    return 1.0 / (1.0 + jnp.exp(-v))


def _a1_body(mes, mee, schpos, sppos, csidx, xwf, ufb,
             c_st, huf_st, hui_st, hsu_st, fcb_st, cnew_st,
             vmeta, idxC, idxP, idxS,
             bufC, bufCP, bufHUf, bufXWf, bufHUi, bufhsU, buf_fc, buf_cn,
             bufb, sem0, sem1, sem2, sem3, sem4, sem5):
    wid = lax.axis_index("s") * NC + lax.axis_index("c")
    pltpu.sync_copy(mes, vmeta)
    es = vmeta[...][0]
    pltpu.sync_copy(mee, vmeta)
    ee = vmeta[...][0]
    pltpu.sync_copy(ufb, bufb)
    nr = (ee - es + (CH * NW - 1)) // (CH * NW)

    def round_body(r, carry):
        start = pl.multiple_of(es + (r * NW + wid) * CH, CH)

        @pl.when(start < ee)
        def _():
            pltpu.sync_copy(schpos.at[pl.ds(start, CH)], idxC)
            pltpu.sync_copy(sppos.at[pl.ds(start, CH)], idxP)
            pltpu.sync_copy(csidx.at[pl.ds(start, CH)], idxS)
            g0 = pltpu.make_async_copy(c_st.at[idxC], bufC, sem0)
            g1 = pltpu.make_async_copy(huf_st.at[idxC], bufHUf, sem1)
            g2 = pltpu.make_async_copy(hui_st.at[idxC], bufHUi, sem2)
            g3 = pltpu.make_async_copy(xwf.at[idxP], bufXWf, sem3)
            g4 = pltpu.make_async_copy(hsu_st.at[idxP], bufhsU, sem4)
            g5 = pltpu.make_async_copy(c_st.at[idxP], bufCP, sem5)
            g0.start(); g1.start(); g2.start(); g3.start(); g4.start(); g5.start()
            g0.wait(); g1.wait(); g2.wait(); g3.wait(); g4.wait(); g5.wait()

            def row_body(i, c2):
                for j in range(F // 16):
                    sl = pl.ds(j * 16, 16)
                    s = bufXWf[i, sl] + bufHUf[i, sl] + bufb[sl]
                    f = 1.0 / (1.0 + jnp.exp(-s))
                    fc = f * bufC[i, sl]
                    buf_fc[i, sl] = fc
                    buf_cn[i, sl] = bufCP[i, sl] + fc
                for j in range(F3 // 16):
                    sl = pl.ds(j * 16, 16)
                    bufhsU[i, sl] = bufhsU[i, sl] + bufHUi[i, sl]
                return c2

            lax.fori_loop(0, CH, row_body, 0)
            s0 = pltpu.make_async_copy(bufhsU, hsu_st.at[idxP], sem0)
            s1 = pltpu.make_async_copy(buf_fc, fcb_st.at[idxP], sem1)
            s2 = pltpu.make_async_copy(buf_cn, cnew_st.at[idxS], sem2)
            s0.start(); s1.start(); s2.start()
            s0.wait(); s1.wait(); s2.wait()
        return carry

    lax.fori_loop(0, nr, round_body, 0)


def _a2_body(mes, mee, csidx, cnew_st, c_st,
             vmeta, idxS, buf, sem0):
    wid = lax.axis_index("s") * NC + lax.axis_index("c")
    pltpu.sync_copy(mes, vmeta)
    es = vmeta[...][0]
    pltpu.sync_copy(mee, vmeta)
    ee = vmeta[...][0]
    nr = (ee - es + (CH * NW - 1)) // (CH * NW)

    def round_body(r, carry):
        start = pl.multiple_of(es + (r * NW + wid) * CH, CH)

        @pl.when(start < ee)
        def _():
            pltpu.sync_copy(csidx.at[pl.ds(start, CH)], idxS)
            g = pltpu.make_async_copy(cnew_st.at[idxS], buf, sem0)
            g.start(); g.wait()
            s = pltpu.make_async_copy(buf, c_st.at[idxS], sem0)
            s.start(); s.wait()
        return carry

    lax.fori_loop(0, nr, round_body, 0)


def _b_body(meta, uiouw, ufw, biou, xw,
            hsu_st, fcb_st, c_st, h_st, hui_st, huf_st,
            meta_s, uiouw_v, ufw_v, biou_v,
            xw_v, hsu_v, fc_v, c_v, h_v, hui_v, huf_v,
            s0, s1, s2, s3, s4, s5, s6):
    cp = pltpu.make_async_copy(meta, meta_s, s0)
    cp.start(); cp.wait()
    cw0 = pltpu.make_async_copy(uiouw, uiouw_v, s0)
    cw1 = pltpu.make_async_copy(ufw, ufw_v, s1)
    cw2 = pltpu.make_async_copy(biou, biou_v, s2)
    cw0.start(); cw1.start(); cw2.start()
    cw0.wait(); cw1.wait(); cw2.wait()
    ns = meta_s[0]
    ne = meta_s[1]
    lvl = meta_s[2]
    ts = ns // TILE
    te = (ne + TILE - 1) // TILE

    def tile_body(t, carry):
        r0 = t * TILE
        sl = pl.ds(r0, TILE)
        g0 = pltpu.make_async_copy(xw.at[sl], xw_v, s0)
        g1 = pltpu.make_async_copy(hsu_st.at[sl], hsu_v, s1)
        g2 = pltpu.make_async_copy(fcb_st.at[sl], fc_v, s2)
        g3 = pltpu.make_async_copy(c_st.at[sl], c_v, s3)
        g4 = pltpu.make_async_copy(h_st.at[sl], h_v, s4)
        g5 = pltpu.make_async_copy(hui_st.at[sl], hui_v, s5)
        g6 = pltpu.make_async_copy(huf_st.at[sl], huf_v, s6)
        g0.start(); g1.start(); g2.start(); g3.start(); g4.start(); g5.start(); g6.start()
        g0.wait(); g1.wait(); g2.wait(); g3.wait(); g4.wait(); g5.wait(); g6.wait()

        rowid = r0 + lax.broadcasted_iota(jnp.int32, (TILE, 1), 0)
        active = (rowid >= ns) & (rowid < ne)
        use_u = (lvl > 0).astype(jnp.float32)
        iou = xw_v[...] + use_u * (hsu_v[...] + biou_v[...])
        i_g = _sigmoid(iou[:, :F])
        o_g = _sigmoid(iou[:, F:2 * F])
        u_g = jnp.tanh(iou[:, 2 * F:])
        cn = i_g * u_g + use_u * fc_v[...]
        hn = o_g * jnp.tanh(cn)
        hui_n = lax.dot_general(hn, uiouw_v[...], (((1,), (1,)), ((), ())),
                                preferred_element_type=jnp.float32)
        huf_n = lax.dot_general(hn, ufw_v[...], (((1,), (1,)), ((), ())),
                                preferred_element_type=jnp.float32)
        c_v[...] = jnp.where(active, cn, c_v[...])
        h_v[...] = jnp.where(active, hn, h_v[...])
        hui_v[...] = jnp.where(active, hui_n, hui_v[...])
        huf_v[...] = jnp.where(active, huf_n, huf_v[...])
        w0 = pltpu.make_async_copy(c_v, c_st.at[sl], s3)
        w1 = pltpu.make_async_copy(h_v, h_st.at[sl], s4)
        w2 = pltpu.make_async_copy(hui_v, hui_st.at[sl], s5)
        w3 = pltpu.make_async_copy(huf_v, huf_st.at[sl], s6)
        w0.start(); w1.start(); w2.start(); w3.start()
        w0.wait(); w1.wait(); w2.wait(); w3.wait()
        return carry

    lax.fori_loop(ts, te, tile_body, 0)


def _perm_body(idx_hbm, src_hbm, out_hbm, idxv, buf, sem0):
    wid = lax.axis_index("s") * NC + lax.axis_index("c")

    def round_body(r, carry):
        start = pl.multiple_of((r * NW + wid) * CH, CH)
        pltpu.sync_copy(idx_hbm.at[pl.ds(start, CH)], idxv)
        g = pltpu.make_async_copy(src_hbm.at[idxv], buf, sem0)
        g.start(); g.wait()
        pltpu.sync_copy(buf, out_hbm.at[pl.ds(start, CH)])
        return carry

    lax.fori_loop(0, NP // CH // NW, round_body, 0)


def _pre_body(x_ref, wiou_ref, wf_ref, biou_ref, bf_ref, xwiou_ref, xwf_ref):
    xv = x_ref[...]
    xwiou_ref[...] = lax.dot_general(
        xv, wiou_ref[...], (((1,), (1,)), ((), ())),
        preferred_element_type=jnp.float32) + biou_ref[...]
    xwf_ref[...] = lax.dot_general(
        xv, wf_ref[...], (((1,), (1,)), ((), ())),
        preferred_element_type=jnp.float32) + bf_ref[...]


def kernel(x, edge_index, edge_feats, edge_types, W_iou_w, W_iou_b,
           U_iou_w, U_iou_b, W_f_w, W_f_b, U_f_w, U_f_b):
    del edge_feats, edge_types
    i32 = jnp.int32
    f32 = jnp.float32
    p = edge_index[0]
    ch = edge_index[1]

    deg = jnp.zeros((N,), i32).at[p].add(1)
    order = jnp.argsort(p, stable=True)
    sp0 = p[order]
    eidx = jnp.arange(E, dtype=i32)
    is_start = jnp.concatenate(
        [jnp.zeros((1,), bool), sp0[1:] != sp0[:-1]])
    gs = lax.cummax(jnp.where(is_start, eidx, 0), axis=0)
    edge_order = jnp.zeros((E,), i32).at[order].set(eidx - gs + 1)

    sp = p
    so = edge_order
    ecnt = jnp.zeros((E + 2,), i32).at[edge_order].add(1)
    pc = ((ecnt + CH - 1) // CH) * CH
    pc = jnp.where(ecnt > 0, pc, 0)
    peoff = jnp.concatenate([jnp.zeros((1,), i32), jnp.cumsum(pc)])
    nperm = jnp.argsort(deg, stable=True)
    ncnt = jnp.zeros((E + 2,), i32).at[deg].add(1)
    noff = jnp.concatenate([jnp.zeros((1,), i32), jnp.cumsum(ncnt)])
    spos = jnp.zeros((N,), i32).at[nperm].set(jnp.arange(N, dtype=i32))

    pidx = peoff[so] + (spos[sp] - noff[so])

    fin = deg[sp] == so
    schpos = jnp.full((PE,), TRASH, i32).at[pidx].set(spos[ch])
    sppos = jnp.full((PE,), TRASH, i32).at[pidx].set(spos[sp])
    csidx = jnp.full((PE,), TRASH, i32).at[pidx].set(
        jnp.where(fin, TRASH, spos[sp]))
    n_iter = jnp.max(deg) + 1

    sc_mesh = plsc.VectorSubcoreMesh(core_axis_name="c", subcore_axis_name="s")
    perm_call = pl.kernel(
        _perm_body,
        mesh=sc_mesh,
        out_type=[jax.ShapeDtypeStruct((NP, F), f32)],
        scratch_types=[
            pltpu.VMEM((CH,), i32),
            pltpu.VMEM((CH, F), f32),
            pltpu.SemaphoreType.DMA,
        ],
    )
    zpad = jnp.zeros((NP - N,), i32)

    res = perm_call(jnp.concatenate([nperm, zpad]), x)
    x_s = res[0] if isinstance(res, (tuple, list)) else res
    biou2 = W_iou_b.reshape(1, F3)
    bf2 = W_f_b.reshape(1, F)
    xw_iou_s, xw_f_s = pl.pallas_call(
        _pre_body,
        grid=(NP // TILE,),
        in_specs=[
            pl.BlockSpec((TILE, F), lambda t: (t, 0)),
            pl.BlockSpec((F3, F), lambda t: (0, 0)),
            pl.BlockSpec((F, F), lambda t: (0, 0)),
            pl.BlockSpec((1, F3), lambda t: (0, 0)),
            pl.BlockSpec((1, F), lambda t: (0, 0)),
        ],
        out_specs=[
            pl.BlockSpec((TILE, F3), lambda t: (t, 0)),
            pl.BlockSpec((TILE, F), lambda t: (t, 0)),
        ],
        out_shape=[
            jax.ShapeDtypeStruct((NP, F3), f32),
            jax.ShapeDtypeStruct((NP, F), f32),
        ],
    )(x_s, W_iou_w, W_f_w, biou2, bf2)

    c_ref = jax.new_ref(jnp.zeros((NP, F), f32))
    hsu_ref = jax.new_ref(jnp.zeros((NP, F3), f32))
    hui_ref = jax.new_ref(jnp.zeros((NP, F3), f32))
    huf_ref = jax.new_ref(jnp.zeros((NP, F), f32))
    h_ref = jax.new_ref(jnp.zeros((NP, F), f32))
    fcb_ref = jax.new_ref(jnp.zeros((NP, F), f32))
    cnew_ref = jax.new_ref(jnp.zeros((NP, F), f32))

    tc_mesh = pltpu.create_tensorcore_mesh("x")

    a1_call = pl.kernel(
        _a1_body,
        mesh=sc_mesh,
        out_type=[],
        scratch_types=[
            pltpu.VMEM((16,), i32),
            pltpu.VMEM((CH,), i32),
            pltpu.VMEM((CH,), i32),
            pltpu.VMEM((CH,), i32),
            pltpu.VMEM((CH, F), f32),
            pltpu.VMEM((CH, F), f32),
            pltpu.VMEM((CH, F), f32),
            pltpu.VMEM((CH, F), f32),
            pltpu.VMEM((CH, F3), f32),
            pltpu.VMEM((CH, F3), f32),
            pltpu.VMEM((CH, F), f32),
            pltpu.VMEM((CH, F), f32),
            pltpu.VMEM((F,), f32),
            pltpu.SemaphoreType.DMA,
            pltpu.SemaphoreType.DMA,
            pltpu.SemaphoreType.DMA,
            pltpu.SemaphoreType.DMA,
            pltpu.SemaphoreType.DMA,
            pltpu.SemaphoreType.DMA,
        ],
    )

    a2_call = pl.kernel(
        _a2_body,
        mesh=sc_mesh,
        out_type=[],
        scratch_types=[
            pltpu.VMEM((16,), i32),
            pltpu.VMEM((CH,), i32),
            pltpu.VMEM((CH, F), f32),
            pltpu.SemaphoreType.DMA,
        ],
    )

    b_call = pl.kernel(
        _b_body,
        mesh=tc_mesh,
        out_type=[],
        scratch_types=[
            pltpu.SMEM((3,), i32),
            pltpu.VMEM((F3, F), f32),
            pltpu.VMEM((F, F), f32),
            pltpu.VMEM((1, F3), f32),
            pltpu.VMEM((TILE, F3), f32),
            pltpu.VMEM((TILE, F3), f32),
            pltpu.VMEM((TILE, F), f32),
            pltpu.VMEM((TILE, F), f32),
            pltpu.VMEM((TILE, F), f32),
            pltpu.VMEM((TILE, F3), f32),
            pltpu.VMEM((TILE, F), f32),
            pltpu.SemaphoreType.DMA,
            pltpu.SemaphoreType.DMA,
            pltpu.SemaphoreType.DMA,
            pltpu.SemaphoreType.DMA,
            pltpu.SemaphoreType.DMA,
            pltpu.SemaphoreType.DMA,
            pltpu.SemaphoreType.DMA,
        ],
    )

    ufb1 = U_f_b.reshape(F)
    biou1 = U_iou_b.reshape(1, F3)

    def body(n, carry):
        es = peoff[n]
        ee = peoff[n + 1]
        ns = noff[n]
        ne = noff[n + 1]
        mes = jnp.full((16,), es, i32)
        mee = jnp.full((16,), ee, i32)
        a1_call(mes, mee, schpos, sppos, csidx, xw_f_s, ufb1,
                c_ref, huf_ref, hui_ref, hsu_ref, fcb_ref, cnew_ref)
        a2_call(mes, mee, csidx, cnew_ref, c_ref)
        meta = jnp.stack([ns, ne, n]).astype(i32)
        b_call(meta, U_iou_w, U_f_w, biou1, xw_iou_s,
               hsu_ref, fcb_ref, c_ref, h_ref, hui_ref, huf_ref)
        return carry

    lax.fori_loop(0, n_iter, body, 0)
    res = perm_call(jnp.concatenate([spos, zpad]), h_ref[...])
    h_out = res[0] if isinstance(res, (tuple, list)) else res
    return h_out[:N]
